# baseline (speedup 1.0000x reference)
import functools

import jax
import jax.numpy as jnp
from jax import lax
from jax.experimental import pallas as pl
from jax.experimental.pallas import tpu as pltpu
from jax.experimental.pallas import tpu_sc as plsc

N_ATOM = 50000
E_EDGE = 800000
A_ANG = 800000
H = 64
G = 50
CUTOFF = 5.0
N_PAD = 51200

_f32 = jnp.float32
_bf16 = jnp.bfloat16
_i32 = jnp.int32

_SC_PARAMS = None


@functools.lru_cache(maxsize=1)
def _mesh():
    return plsc.VectorSubcoreMesh(core_axis_name="c", subcore_axis_name="s",
                                  num_cores=2, num_subcores=16)


_NW = 32
_CP = dict(use_tc_tiling_on_sc=False, needs_layout_passes=False)


def _silu(x):
    return x * jax.nn.sigmoid(x)


def _make_gather(n_rows, d):
    n_chunks = n_rows // 128

    @functools.partial(
        pl.kernel,
        mesh=_mesh(),
        compiler_params=pltpu.CompilerParams(**_CP),
        out_type=jax.ShapeDtypeStruct((n_rows, d), _f32),
        scratch_types=[
            pltpu.VMEM((128,), _i32),
            pltpu.VMEM((128, d), _f32),
            pltpu.SemaphoreType.DMA,
        ],
        name=f"sc_gather_{n_rows}_{d}",
    )
    def k(src_hbm, idx_hbm, out_hbm, idx_v, rows_v, sem):
        w = lax.axis_index("s") * 2 + lax.axis_index("c")
        nj = (n_chunks - w + _NW - 1) // _NW

        def body(i, _):
            base = pl.multiple_of((w + i * _NW) * 128, 128)
            pltpu.sync_copy(idx_hbm.at[pl.ds(base, 128)], idx_v)
            pltpu.async_copy(src_hbm.at[idx_v], rows_v, sem).wait()
            pltpu.sync_copy(rows_v, out_hbm.at[pl.ds(base, 128)])
            return 0

        lax.fori_loop(0, nj, body, 0)

    return k


def _make_gather2sum(n_rows, d):
    bpw = n_rows // _NW
    assert bpw % 8 == 0
    nfull, tail = divmod(bpw, 256)
    npair, leftover = divmod(nfull, 2)
    assert npair >= 2

    @functools.partial(
        pl.kernel,
        mesh=_mesh(),
        compiler_params=pltpu.CompilerParams(**_CP),
        out_type=jax.ShapeDtypeStruct((n_rows, d), _f32),
        scratch_types=[
            pltpu.VMEM((2, 256), _i32),
            pltpu.VMEM((2, 256), _i32),
            pltpu.VMEM((2, 256, d), _f32),
            pltpu.VMEM((2, 256, d), _f32),
        ] + [pltpu.SemaphoreType.DMA] * 6,
        name=f"sc_g2s_{n_rows}_{d}",
    )
    def k(srca_hbm, srcb_hbm, ia_hbm, ib_hbm, out_hbm,
          ia_v, ib_v, ra_v, rb_v, si0, si1, sg0, sg1, so0, so1):
        w = lax.axis_index("s") * 2 + lax.axis_index("c")
        wbase = pl.multiple_of(w * bpw, 8)
        si = (si0, si1)
        sg = (sg0, sg1)
        so = (so0, so1)

        def fire_idx(kd, b):
            base = pl.multiple_of(wbase + kd * 256, 8)
            pltpu.async_copy(ia_hbm.at[pl.ds(base, 256)], ia_v.at[b], si[b])
            pltpu.async_copy(ib_hbm.at[pl.ds(base, 256)], ib_v.at[b], si[b])

        def drain_idx(b):
            pltpu.make_async_copy(ia_hbm.at[pl.ds(0, 256)], ia_v.at[b],
                                  si[b]).wait()
            pltpu.make_async_copy(ib_hbm.at[pl.ds(0, 256)], ib_v.at[b],
                                  si[b]).wait()

        def fire_gathers(b):
            for off in (0, 128):
                sl = pl.ds(off, 128)
                pltpu.async_copy(srca_hbm.at[ia_v.at[b, sl]],
                                 ra_v.at[b, sl], sg[b])
                pltpu.async_copy(srcb_hbm.at[ib_v.at[b, sl]],
                                 rb_v.at[b, sl], sg[b])

        def drain_gathers(b):
            pltpu.make_async_copy(out_hbm.at[pl.ds(0, 256)], ra_v.at[b],
                                  sg[b]).wait()
            pltpu.make_async_copy(out_hbm.at[pl.ds(0, 256)], rb_v.at[b],
                                  sg[b]).wait()

        def add_rows(b, nr):
            def rbody(r, _):
                for q in range(d // 16):
                    sl = pl.ds(q * 16, 16)
                    ra_v[b, r, sl] = ra_v[b, r, sl] + rb_v[b, r, sl]
                return 0

            lax.fori_loop(0, nr, rbody, 0)

        def fire_out(kd, b):
            pltpu.async_copy(ra_v.at[b],
                             out_hbm.at[pl.ds(wbase + kd * 256, 256)], so[b])

        def drain_out(b):
            pltpu.make_async_copy(ra_v.at[b], out_hbm.at[pl.ds(0, 256)],
                                  so[b]).wait()

        fire_idx(0, 0)

        def pair_body(p, _):
            for b in (0, 1):
                kk = 2 * p + b

                @pl.when(kk + 1 < nfull)
                def _():
                    fire_idx(kk + 1, 1 - b)

                @pl.when(p >= 1)
                def _():
                    drain_out(b)

                drain_idx(b)
                fire_gathers(b)
                drain_gathers(b)
                add_rows(b, 256)
                fire_out(kk, b)
            return 0

        lax.fori_loop(0, npair, pair_body, 0)

        if leftover:
            drain_out(0)
            drain_idx(0)
            fire_gathers(0)
            drain_gathers(0)
            add_rows(0, 256)
            fire_out(nfull - 1, 0)

        if tail:
            drain_out(1)
            tbase = pl.multiple_of(wbase + nfull * 256, 8)
            pltpu.sync_copy(ia_hbm.at[pl.ds(tbase, tail)],
                            ia_v.at[1, pl.ds(0, tail)])
            pltpu.sync_copy(ib_hbm.at[pl.ds(tbase, tail)],
                            ib_v.at[1, pl.ds(0, tail)])
            for off in range(0, tail, 128):
                sz = min(128, tail - off)
                sl = pl.ds(off, sz)
                pltpu.async_copy(srca_hbm.at[ia_v.at[1, sl]],
                                 ra_v.at[1, sl], sg[1])
                pltpu.async_copy(srcb_hbm.at[ib_v.at[1, sl]],
                                 rb_v.at[1, sl], sg[1])
                pltpu.make_async_copy(out_hbm.at[pl.ds(0, sz)],
                                      ra_v.at[1, sl], sg[1]).wait()
                pltpu.make_async_copy(out_hbm.at[pl.ds(0, sz)],
                                      rb_v.at[1, sl], sg[1]).wait()
            add_rows(1, tail)
            pltpu.sync_copy(ra_v.at[1, pl.ds(0, tail)],
                            out_hbm.at[pl.ds(tbase, tail)])
        else:
            drain_out(1)
        drain_out(0)

    return k


def _adjust_idx(idx_v, obase, r_size):
    for row in (0, 1):
        for v in range(8):
            sl = pl.ds(v * 16, 16)
            iv = idx_v[row, sl] - obase
            ok = (iv >= 0) & (iv < r_size)
            idx_v[row, sl] = jnp.where(ok, iv, r_size)


def _scal_from_vmem(rng_v, pos):
    vec = plsc.load_gather(rng_v, [jnp.zeros((16,), _i32) + pos])
    return jnp.max(vec)


def _zero_acc(acc, zero_v, s, zc):
    for tb in (s * zc, (s + 16) * zc):
        for off in range(0, zc, 128):
            sz = min(128, zc - off)
            pltpu.sync_copy(zero_v.at[pl.ds(0, sz)],
                            acc.at[pl.ds(tb + off, sz)])


def _copy_out_acc(acc, out_hbm, obase, s, zc):
    pltpu.sync_copy(acc.at[pl.ds(s * zc, zc)],
                    out_hbm.at[pl.ds(obase + s * zc, zc)])
    pltpu.sync_copy(acc.at[pl.ds((s + 16) * zc, zc)],
                    out_hbm.at[pl.ds(obase + (s + 16) * zc, zc)])


def _make_scatter_add(m_rows, d, r_size, n_half):
    zc = r_size // 32

    @functools.partial(
        pl.kernel,
        mesh=_mesh(),
        compiler_params=pltpu.CompilerParams(**_CP),
        out_type=jax.ShapeDtypeStruct((n_half * r_size, d), _f32),
        scratch_types=[
            pltpu.VMEM_SHARED((r_size + 16, d), _f32),
            pltpu.VMEM((256, d), _f32),
            pltpu.VMEM((2, 128), _i32),
            pltpu.VMEM((128, d), _f32),
            pltpu.VMEM((2 * n_half,), _i32),
            pltpu.SemaphoreType.DMA,
        ],
        name=f"sc_segsum_{m_rows}_{r_size}_{n_half}",
    )
    def k(vals_hbm, idx2_hbm, ranges_hbm, zeros_hbm, out_hbm,
          acc, vals_v, idx_v, zero_v, rng_v, sem):
        c = lax.axis_index("c")
        s = lax.axis_index("s")
        pltpu.sync_copy(ranges_hbm, rng_v)
        pltpu.sync_copy(zeros_hbm.at[pl.ds(0, 128)], zero_v)
        nh_mine = (n_half - c + 1) // 2

        def half_body(p, _):
            h = c + 2 * p
            obase = h * r_size
            _zero_acc(acc, zero_v, s, zc)
            plsc.subcore_barrier()
            lo = pl.multiple_of(_scal_from_vmem(rng_v, 2 * h), 256)
            hi = _scal_from_vmem(rng_v, 2 * h + 1)
            nch = (hi - lo) // 256
            nj = (nch - s + 15) // 16

            def chunk_body(i, _):
                base = pl.multiple_of(lo + (s + i * 16) * 256, 256)
                cp_i = pltpu.async_copy(idx2_hbm.at[pl.ds(base // 128, 2)],
                                        idx_v, sem)
                cp_v = pltpu.async_copy(vals_hbm.at[pl.ds(base, 256)],
                                        vals_v, sem)
                cp_i.wait()
                cp_v.wait()
                _adjust_idx(idx_v, obase, r_size)
                pltpu.sync_copy(vals_v.at[pl.ds(0, 128)],
                                acc.at[idx_v.at[0]], add=True)
                pltpu.sync_copy(vals_v.at[pl.ds(128, 128)],
                                acc.at[idx_v.at[1]], add=True)
                return 0

            lax.fori_loop(0, nj, chunk_body, 0)
            plsc.subcore_barrier()
            _copy_out_acc(acc, out_hbm, obase, s, zc)
            plsc.subcore_barrier()
            return 0

        lax.fori_loop(0, nh_mine, half_body, 0)

    return k


def _make_angle_msg(m_rows, d, r_size, n_half):
    zc = r_size // 32

    @functools.partial(
        pl.kernel,
        mesh=_mesh(),
        compiler_params=pltpu.CompilerParams(**_CP),
        out_type=jax.ShapeDtypeStruct((n_half * r_size, d), _bf16),
        scratch_types=[
            pltpu.VMEM_SHARED((r_size + 16, d), _bf16),
            pltpu.VMEM((256, d), _bf16),
            pltpu.VMEM((256, d), _bf16),
            pltpu.VMEM((256, d), _bf16),
            pltpu.VMEM((2, 256), _i32),
            pltpu.VMEM((2, 256), _i32),
            pltpu.VMEM((2, 2, 128), _i32),
            pltpu.VMEM((128, d), _bf16),
            pltpu.VMEM((2 * n_half,), _i32),
            pltpu.SemaphoreType.DMA,
            pltpu.SemaphoreType.DMA,
            pltpu.SemaphoreType.DMA,
            pltpu.SemaphoreType.DMA,
            pltpu.SemaphoreType.DMA,
        ],
        name=f"sc_angle_{m_rows}_{r_size}_{n_half}",
    )
    def k(bond_hbm, t_hbm, g_hbm, exp_hbm, ji_hbm, red2_hbm, ranges_hbm,
          zeros_hbm, out_hbm,
          acc, b_v, t_v, g_v, ei_v, ji_v, red_v, zero_v, rng_v,
          si0, si1, sg0, sg1, ssc):
        c = lax.axis_index("c")
        s = lax.axis_index("s")
        pltpu.sync_copy(ranges_hbm, rng_v)
        pltpu.sync_copy(zeros_hbm.at[pl.ds(0, 128)], zero_v)
        nh_mine = (n_half - c + 1) // 2
        si = (si0, si1)

        def fire_idx(base, b):
            pltpu.async_copy(exp_hbm.at[pl.ds(base, 256)], ei_v.at[b], si[b])
            pltpu.async_copy(ji_hbm.at[pl.ds(base, 256)], ji_v.at[b], si[b])
            pltpu.async_copy(red2_hbm.at[pl.ds(base // 128, 2)],
                             red_v.at[b], si[b])

        def drain_idx(b):
            pltpu.make_async_copy(exp_hbm.at[pl.ds(0, 256)], ei_v.at[b],
                                  si[b]).wait()
            pltpu.make_async_copy(ji_hbm.at[pl.ds(0, 256)], ji_v.at[b],
                                  si[b]).wait()
            pltpu.make_async_copy(red2_hbm.at[pl.ds(0, 2)], red_v.at[b],
                                  si[b]).wait()

        def half_body(p, sc_live):
            h = c + 2 * p
            obase = h * r_size
            _zero_acc(acc, zero_v, s, zc)
            plsc.subcore_barrier()
            lo = pl.multiple_of(_scal_from_vmem(rng_v, 2 * h), 256)
            hi = _scal_from_vmem(rng_v, 2 * h + 1)
            nch = (hi - lo) // 256
            nj = (nch - s + 15) // 16

            @pl.when(nj > 0)
            def _():
                fire_idx(lo + s * 256, 0)

            def chunk_body(i, sc_live):
                b = jax.lax.rem(i, 2)
                for bb in (0, 1):
                    @pl.when(b == bb)
                    def _():
                        process(i, bb)
                return sc_live

            def process(i, bb):
                base = pl.multiple_of(lo + (s + i * 16) * 256, 256)
                drain_idx(bb)

                @pl.when(i + 1 < nj)
                def _():
                    fire_idx(lo + (s + (i + 1) * 16) * 256, 1 - bb)

                @pl.when(i > 0)
                def _():
                    pltpu.make_async_copy(g_hbm.at[pl.ds(0, 256)],
                                          b_v, ssc).wait()
                for off, sgx in ((0, sg0), (128, sg1)):
                    sl = pl.ds(off, 128)
                    pltpu.async_copy(bond_hbm.at[ei_v.at[bb, sl]],
                                     b_v.at[sl], sgx)
                    pltpu.async_copy(t_hbm.at[ji_v.at[bb, sl]], t_v.at[sl],
                                     sgx)
                    pltpu.async_copy(g_hbm.at[pl.ds(base + off, 128)],
                                     g_v.at[sl], sgx)
                _adjust_idx(red_v.at[bb], obase, r_size)
                for off, sgx, rr in ((0, sg0, 0), (128, sg1, 1)):
                    sl = pl.ds(off, 128)
                    for ref in (b_v, t_v, g_v):
                        pltpu.make_async_copy(g_hbm.at[pl.ds(0, 128)],
                                              ref.at[sl], sgx).wait()

                    def mul_body(r, _):
                        for q in range(d // 32):
                            ql = pl.ds(q * 32, 32)
                            b_v[r, ql] = (b_v[r, ql] * g_v[r, ql]
                                          * t_v[r, ql])
                        return 0

                    lax.fori_loop(off, off + 128, mul_body, 0)
                    pltpu.async_copy(b_v.at[sl], acc.at[red_v.at[bb, rr]],
                                     ssc, add=True)

            _ = lax.fori_loop(0, nj, chunk_body, 0)

            @pl.when(nj > 0)
            def _():
                pltpu.make_async_copy(g_hbm.at[pl.ds(0, 256)], b_v,
                                      ssc).wait()
            plsc.subcore_barrier()
            _copy_out_acc(acc, out_hbm, obase, s, zc)
            plsc.subcore_barrier()
            return sc_live

        lax.fori_loop(0, nh_mine, half_body, 0)

    return k


_R_E, _NH_E = 16000, 50
_R_N, _NH_N = 25600, 2


@functools.lru_cache(maxsize=1)
def _sc_kernels():
    return {
        "gather_N": _make_gather(N_PAD, H),
        "g2s_E": _make_gather2sum(E_EDGE, H),
        "scatter_N": _make_scatter_add(E_EDGE, H, _R_N, _NH_N),
        "angle_E": _make_angle_msg(A_ANG, H, _R_E, _NH_E),
    }


def _sorted_ranges(red, r_size, n_half):
    bounds = jnp.arange(n_half + 1, dtype=_i32) * r_size
    ss = jnp.searchsorted(red, bounds).astype(_i32)
    lo = (ss[:-1] // 256) * 256
    hi = jnp.minimum(((ss[1:] + 255) // 256) * 256, red.shape[0])
    return jnp.stack([lo, hi], axis=1).reshape(-1).astype(_i32)


_B_E = 8000
_B_N = 6400


def _row_spec(b, d):
    return pl.BlockSpec((b, d), lambda i: (i, 0))


def _w_spec(*shape):
    return pl.BlockSpec(shape, lambda i: (0,) * len(shape))


def _basis_T(x_ref, dmax):
    x_row = x_ref[...].reshape(1, -1)
    cen = (lax.broadcasted_iota(_i32, (G, 1), 0).astype(_f32)
           * (dmax / (G - 1)))
    return jnp.exp(-((cen - x_row) ** 2) * 5.0)


def _dotT(bT, w):
    return lax.dot_general(bT, w, (((0,), (0,)), ((), ())),
                           preferred_element_type=_f32)


def _pq_body(emb_ref, w0i, w0j, p_ref, q_ref):
    emb = emb_ref[...]
    p_ref[...] = jnp.dot(emb, w0i[...], preferred_element_type=_f32)
    q_ref[...] = jnp.dot(emb, w0j[...], preferred_element_type=_f32)


def _embed_body(dist_ref, gsum_ref, w0r, b0, w1, b1, out_ref):
    rbT = _basis_T(dist_ref, CUTOFF)
    z = _dotT(rbT, w0r[...]) + gsum_ref[...]
    z = _silu(z + b0[...])
    out_ref[...] = _silu(jnp.dot(z, w1[...], preferred_element_type=_f32)
                         + b1[...])


def _basis_body(ang_ref, w_sbf, out_ref):
    out_ref[...] = _dotT(_basis_T(ang_ref, 3.14),
                         w_sbf[...]).astype(_bf16)


def _t_body(bond_ref, wkj, bkj, wim, bim, tkj_ref, tim_ref, bbf_ref):
    bond = bond_ref[...]
    bbf_ref[...] = bond.astype(_bf16)
    tkj_ref[...] = _silu(jnp.dot(bond, wkj[...],
                                 preferred_element_type=_f32)
                         + bkj[...]).astype(_bf16)
    tim_ref[...] = _silu(jnp.dot(bond, wim[...],
                                 preferred_element_type=_f32)
                         + bim[...]).astype(_bf16)


def _bond_update_body(bond_ref, akj_ref, aim_ref, w_out, b_out, wr1, br1,
                      wr2, br2, w_b, b_b, b2_ref, tmp_ref):
    x = (bond_ref[...] + akj_ref[...].astype(_f32)
         + aim_ref[...].astype(_f32))
    b2 = _silu(jnp.dot(x, w_out[...], preferred_element_type=_f32) + b_out[...])
    b2 = b2 + _silu(jnp.dot(b2, wr1[...], preferred_element_type=_f32)
                    + br1[...])
    b2 = b2 + _silu(jnp.dot(b2, wr2[...], preferred_element_type=_f32)
                    + br2[...])
    b2_ref[...] = b2
    tmp_ref[...] = _silu(jnp.dot(b2, w_b[...], preferred_element_type=_f32)
                         + b_b[...])


def _atom_update_body(atom_ref, agg_ref, hid_ref, w_a, b_a, w_i, w_j,
                      atom_out, hid_out, ai_out, aj_out):
    an = _silu(jnp.dot(atom_ref[...] + agg_ref[...], w_a[...],
                       preferred_element_type=_f32) + b_a[...])
    atom_out[...] = an
    hid_out[...] = hid_ref[...] + an
    ai_out[...] = jnp.dot(an, w_i[...], preferred_element_type=_f32)
    aj_out[...] = jnp.dot(an, w_j[...], preferred_element_type=_f32)


def _bond_atom_body(b2_ref, gsum_ref, w_bb, b_a2b, out_ref):
    out_ref[...] = _silu(jnp.dot(b2_ref[...], w_bb[...],
                                 preferred_element_type=_f32)
                         + gsum_ref[...] + b_a2b[...])


def _readout_body(hg_ref, w0, b0, w1, b1, w2, b2, ow, ob, out_ref):
    h = hg_ref[...]
    h = _silu(jnp.dot(h, w0[...], preferred_element_type=_f32) + b0[...])
    h = _silu(jnp.dot(h, w1[...], preferred_element_type=_f32) + b1[...])
    h = _silu(jnp.dot(h, w2[...], preferred_element_type=_f32) + b2[...])
    out_ref[...] = jnp.dot(h, ow[...], preferred_element_type=_f32) + ob[...]


def _tc_call(body, grid, in_specs, out_specs, out_shapes, name):
    return pl.pallas_call(
        body, grid=(grid,), in_specs=in_specs, out_specs=out_specs,
        out_shape=out_shapes, name=name)


def kernel(atom_features, id_i, id_j, dist, angle_kj, angle_im, kj_exp,
           kj_ji_exp, kj_red, im_exp, im_ji_exp, im_red, reduce_idx, params):
    p = params
    sck = _sc_kernels()
    _gather_N = sck["gather_N"]
    _g2s_E = sck["g2s_E"]
    _scatter_N = sck["scatter_N"]
    _angle_E = sck["angle_E"]
    npad = N_PAD - N_ATOM
    feat_p = jnp.pad(atom_features.astype(_i32), (0, npad))
    reduce_p = jnp.pad(reduce_idx.astype(_i32), (0, npad))
    id_i = id_i.astype(_i32)
    id_j = id_j.astype(_i32)
    id_i2 = id_i.reshape(E_EDGE // 128, 128)
    zeros_z = jnp.zeros((128, H), _f32)
    zeros_b = jnp.zeros((128, H), _bf16)
    rng_n = jnp.array([0, E_EDGE, 0, E_EDGE], _i32)

    dist2 = dist.reshape(E_EDGE // _B_E, 1, _B_E)
    akj2 = angle_kj.reshape(A_ANG // _B_E, 1, _B_E)
    aim2 = angle_im.reshape(A_ANG // _B_E, 1, _B_E)

    grid_e = E_EDGE // _B_E
    grid_n = N_PAD // _B_N
    row_e = _row_spec(_B_E, H)
    row_n = _row_spec(_B_N, H)
    scal_e = pl.BlockSpec((1, 1, _B_E), lambda i: (i, 0, 0))
    wh = _w_spec(H, H)
    wb = _w_spec(1, H)
    wg = _w_spec(G, H)
    sde = jax.ShapeDtypeStruct((E_EDGE, H), _f32)
    sdeb = jax.ShapeDtypeStruct((E_EDGE, H), _bf16)
    sdn = jax.ShapeDtypeStruct((N_PAD, H), _f32)

    def b2d(b):
        return b.reshape(1, H)

    atom_emb = _gather_N(p["atom_table"], feat_p)

    w0 = p["emb_W0"]
    pe, qe = _tc_call(_pq_body, grid_n, [row_n, wh, wh],
                      [row_n, row_n], [sdn, sdn], "tc_pq")(
        atom_emb, w0[:H], w0[H:2 * H])
    gsum0 = _g2s_E(pe, qe, id_i, id_j)
    bond = _tc_call(
        _embed_body, grid_e, [scal_e, row_e, wg, wb, wh, wb],
        _row_spec(_B_E, H), sde, "tc_embed")(
        dist2, gsum0, w0[2 * H:], b2d(p["emb_b0"]), p["emb_W1"],
        b2d(p["emb_b1"]))

    atom = atom_emb
    hidden = atom_emb

    kj_red = kj_red.astype(_i32)
    im_red = im_red.astype(_i32)
    rng_kj = _sorted_ranges(kj_red, _R_E, _NH_E)
    rng_im = _sorted_ranges(im_red, _R_E, _NH_E)
    kj_red2 = kj_red.reshape(A_ANG // 128, 128)
    im_red2 = im_red.reshape(A_ANG // 128, 128)

    for lay in p["layers"]:
        g_kj = _tc_call(_basis_body, grid_e, [scal_e, wg],
                        _row_spec(_B_E, H), sdeb, "tc_basis")(
            akj2, lay["W_sbf_kj"])
        g_im = _tc_call(_basis_body, grid_e, [scal_e, wg],
                        _row_spec(_B_E, H), sdeb, "tc_basis")(
            aim2, lay["W_sbf_im"])
        t_kj, t_im, bond_bf = _tc_call(
            _t_body, grid_e, [row_e, wh, wb, wh, wb],
            [row_e, row_e, row_e], [sdeb, sdeb, sdeb], "tc_tmsg")(
            bond, lay["W_ji_kj"], b2d(lay["b_ji_kj"]),
            lay["W_ji_im"], b2d(lay["b_ji_im"]))

        agg_kj = _angle_E(bond_bf, t_kj, g_kj, kj_exp.astype(_i32),
                          kj_ji_exp.astype(_i32), kj_red2, rng_kj, zeros_b)
        agg_im = _angle_E(bond_bf, t_im, g_im, im_exp.astype(_i32),
                          im_ji_exp.astype(_i32), im_red2, rng_im, zeros_b)

        res = lay["res"]
        b2, tmp = _tc_call(
            _bond_update_body, grid_e,
            [row_e, row_e, row_e, wh, wb, wh, wb, wh, wb, wh, wb],
            [row_e, row_e], [sde, sde], "tc_bond_update")(
            bond, agg_kj, agg_im, lay["W_out"], b2d(lay["b_out"]),
            res[0]["W"], b2d(res[0]["b"]), res[1]["W"], b2d(res[1]["b"]),
            lay["W_b"], b2d(lay["b_b"]))

        agg_a = _scatter_N(tmp, id_i2, rng_n, zeros_z)

        atom, hidden, ai, aj = _tc_call(
            _atom_update_body, grid_n,
            [row_n, row_n, row_n, wh, wb, wh, wh],
            [row_n, row_n, row_n, row_n], [sdn, sdn, sdn, sdn],
            "tc_atom_update")(
            atom, agg_a, hidden, lay["W_a"], b2d(lay["b_a"]),
            lay["W_i"], lay["W_j"])

        gs = _g2s_E(ai, aj, id_i, id_j)
        bond = _tc_call(
            _bond_atom_body, grid_e, [row_e, row_e, wh, wb],
            _row_spec(_B_E, H), sde, "tc_bond_atom")(
            b2, gs, lay["W_bb"], b2d(lay["b_a2b"]))

    hg = _gather_N(hidden, reduce_p)
    ro = p["readout"]
    ow = jnp.pad(p["out_W"], ((0, 0), (0, 128 - p["out_W"].shape[1])))
    ob = jnp.pad(p["out_b"], (0, 128 - p["out_b"].shape[0])).reshape(1, 128)
    out = _tc_call(
        _readout_body, grid_n,
        [row_n, wh, wb, wh, wb, wh, wb, _w_spec(H, 128), _w_spec(1, 128)],
        _row_spec(_B_N, 128), jax.ShapeDtypeStruct((N_PAD, 128), _f32),
        "tc_readout")(
        hg, ro[0]["W"], b2d(ro[0]["b"]), ro[1]["W"], b2d(ro[1]["b"]),
        ro[2]["W"], b2d(ro[2]["b"]), ow, ob)
    return out[:N_ATOM, :p["out_W"].shape[1]]


# speedup vs baseline: 1.0902x; 1.0902x over previous
import functools

import jax
import jax.numpy as jnp
from jax import lax
from jax.experimental import pallas as pl
from jax.experimental.pallas import tpu as pltpu
from jax.experimental.pallas import tpu_sc as plsc

N_ATOM = 50000
E_EDGE = 800000
A_ANG = 800000
H = 64
G = 50
CUTOFF = 5.0
N_PAD = 51200

_f32 = jnp.float32
_i32 = jnp.int32

_SC_PARAMS = None


@functools.lru_cache(maxsize=1)
def _mesh():
    return plsc.VectorSubcoreMesh(core_axis_name="c", subcore_axis_name="s",
                                  num_cores=2, num_subcores=16)


_NW = 32
_CP = dict(use_tc_tiling_on_sc=False, needs_layout_passes=False)


def _silu(x):
    return x * jax.nn.sigmoid(x)


def _make_gather(n_rows, d):
    n_chunks = n_rows // 128

    @functools.partial(
        pl.kernel,
        mesh=_mesh(),
        compiler_params=pltpu.CompilerParams(**_CP),
        out_type=jax.ShapeDtypeStruct((n_rows, d), _f32),
        scratch_types=[
            pltpu.VMEM((128,), _i32),
            pltpu.VMEM((128, d), _f32),
            pltpu.SemaphoreType.DMA,
        ],
        name=f"sc_gather_{n_rows}_{d}",
    )
    def k(src_hbm, idx_hbm, out_hbm, idx_v, rows_v, sem):
        w = lax.axis_index("s") * 2 + lax.axis_index("c")
        nj = (n_chunks - w + _NW - 1) // _NW

        def body(i, _):
            base = pl.multiple_of((w + i * _NW) * 128, 128)
            pltpu.sync_copy(idx_hbm.at[pl.ds(base, 128)], idx_v)
            pltpu.async_copy(src_hbm.at[idx_v], rows_v, sem).wait()
            pltpu.sync_copy(rows_v, out_hbm.at[pl.ds(base, 128)])
            return 0

        lax.fori_loop(0, nj, body, 0)

    return k


def _make_gather2sum(n_rows, d):
    bpw = n_rows // _NW
    assert bpw % 8 == 0
    nfull, tail = divmod(bpw, 256)
    npair, leftover = divmod(nfull, 2)
    assert npair >= 2

    @functools.partial(
        pl.kernel,
        mesh=_mesh(),
        compiler_params=pltpu.CompilerParams(**_CP),
        out_type=jax.ShapeDtypeStruct((n_rows, d), _f32),
        scratch_types=[
            pltpu.VMEM((2, 256), _i32),
            pltpu.VMEM((2, 256), _i32),
            pltpu.VMEM((2, 256, d), _f32),
            pltpu.VMEM((2, 256, d), _f32),
        ] + [pltpu.SemaphoreType.DMA] * 6,
        name=f"sc_g2s_{n_rows}_{d}",
    )
    def k(srca_hbm, srcb_hbm, ia_hbm, ib_hbm, out_hbm,
          ia_v, ib_v, ra_v, rb_v, si0, si1, sg0, sg1, so0, so1):
        w = lax.axis_index("s") * 2 + lax.axis_index("c")
        wbase = pl.multiple_of(w * bpw, 8)
        si = (si0, si1)
        sg = (sg0, sg1)
        so = (so0, so1)

        def fire_idx(kd, b):
            base = pl.multiple_of(wbase + kd * 256, 8)
            pltpu.async_copy(ia_hbm.at[pl.ds(base, 256)], ia_v.at[b], si[b])
            pltpu.async_copy(ib_hbm.at[pl.ds(base, 256)], ib_v.at[b], si[b])

        def drain_idx(b):
            pltpu.make_async_copy(ia_hbm.at[pl.ds(0, 256)], ia_v.at[b],
                                  si[b]).wait()
            pltpu.make_async_copy(ib_hbm.at[pl.ds(0, 256)], ib_v.at[b],
                                  si[b]).wait()

        def fire_gathers(b):
            for off in (0, 128):
                sl = pl.ds(off, 128)
                pltpu.async_copy(srca_hbm.at[ia_v.at[b, sl]],
                                 ra_v.at[b, sl], sg[b])
                pltpu.async_copy(srcb_hbm.at[ib_v.at[b, sl]],
                                 rb_v.at[b, sl], sg[b])

        def drain_gathers(b):
            pltpu.make_async_copy(out_hbm.at[pl.ds(0, 256)], ra_v.at[b],
                                  sg[b]).wait()
            pltpu.make_async_copy(out_hbm.at[pl.ds(0, 256)], rb_v.at[b],
                                  sg[b]).wait()

        def add_rows(b, nr):
            def rbody(r, _):
                for q in range(d // 16):
                    sl = pl.ds(q * 16, 16)
                    ra_v[b, r, sl] = ra_v[b, r, sl] + rb_v[b, r, sl]
                return 0

            lax.fori_loop(0, nr, rbody, 0)

        def fire_out(kd, b):
            pltpu.async_copy(ra_v.at[b],
                             out_hbm.at[pl.ds(wbase + kd * 256, 256)], so[b])

        def drain_out(b):
            pltpu.make_async_copy(ra_v.at[b], out_hbm.at[pl.ds(0, 256)],
                                  so[b]).wait()

        fire_idx(0, 0)

        def pair_body(p, _):
            for b in (0, 1):
                kk = 2 * p + b

                @pl.when(kk + 1 < nfull)
                def _():
                    fire_idx(kk + 1, 1 - b)

                @pl.when(p >= 1)
                def _():
                    drain_out(b)

                drain_idx(b)
                fire_gathers(b)
                drain_gathers(b)
                add_rows(b, 256)
                fire_out(kk, b)
            return 0

        lax.fori_loop(0, npair, pair_body, 0)

        if leftover:
            drain_out(0)
            drain_idx(0)
            fire_gathers(0)
            drain_gathers(0)
            add_rows(0, 256)
            fire_out(nfull - 1, 0)

        if tail:
            drain_out(1)
            tbase = pl.multiple_of(wbase + nfull * 256, 8)
            pltpu.sync_copy(ia_hbm.at[pl.ds(tbase, tail)],
                            ia_v.at[1, pl.ds(0, tail)])
            pltpu.sync_copy(ib_hbm.at[pl.ds(tbase, tail)],
                            ib_v.at[1, pl.ds(0, tail)])
            for off in range(0, tail, 128):
                sz = min(128, tail - off)
                sl = pl.ds(off, sz)
                pltpu.async_copy(srca_hbm.at[ia_v.at[1, sl]],
                                 ra_v.at[1, sl], sg[1])
                pltpu.async_copy(srcb_hbm.at[ib_v.at[1, sl]],
                                 rb_v.at[1, sl], sg[1])
                pltpu.make_async_copy(out_hbm.at[pl.ds(0, sz)],
                                      ra_v.at[1, sl], sg[1]).wait()
                pltpu.make_async_copy(out_hbm.at[pl.ds(0, sz)],
                                      rb_v.at[1, sl], sg[1]).wait()
            add_rows(1, tail)
            pltpu.sync_copy(ra_v.at[1, pl.ds(0, tail)],
                            out_hbm.at[pl.ds(tbase, tail)])
        else:
            drain_out(1)
        drain_out(0)

    return k


def _adjust_idx(idx_v, obase, r_size):
    for row in (0, 1):
        for v in range(8):
            sl = pl.ds(v * 16, 16)
            iv = idx_v[row, sl] - obase
            ok = (iv >= 0) & (iv < r_size)
            idx_v[row, sl] = jnp.where(ok, iv, r_size)


def _scal_from_vmem(rng_v, pos):
    vec = plsc.load_gather(rng_v, [jnp.zeros((16,), _i32) + pos])
    return jnp.max(vec)


def _zero_acc(acc, zero_v, s, zc):
    for tb in (s * zc, (s + 16) * zc):
        for off in range(0, zc, 128):
            sz = min(128, zc - off)
            pltpu.sync_copy(zero_v.at[pl.ds(0, sz)],
                            acc.at[pl.ds(tb + off, sz)])


def _copy_out_acc(acc, out_hbm, obase, s, zc):
    pltpu.sync_copy(acc.at[pl.ds(s * zc, zc)],
                    out_hbm.at[pl.ds(obase + s * zc, zc)])
    pltpu.sync_copy(acc.at[pl.ds((s + 16) * zc, zc)],
                    out_hbm.at[pl.ds(obase + (s + 16) * zc, zc)])


def _make_scatter_add(m_rows, d, r_size, n_half):
    zc = r_size // 32

    @functools.partial(
        pl.kernel,
        mesh=_mesh(),
        compiler_params=pltpu.CompilerParams(**_CP),
        out_type=jax.ShapeDtypeStruct((n_half * r_size, d), _f32),
        scratch_types=[
            pltpu.VMEM_SHARED((r_size + 16, d), _f32),
            pltpu.VMEM((256, d), _f32),
            pltpu.VMEM((2, 128), _i32),
            pltpu.VMEM((128, d), _f32),
            pltpu.VMEM((2 * n_half,), _i32),
            pltpu.SemaphoreType.DMA,
        ],
        name=f"sc_segsum_{m_rows}_{r_size}_{n_half}",
    )
    def k(vals_hbm, idx2_hbm, ranges_hbm, zeros_hbm, out_hbm,
          acc, vals_v, idx_v, zero_v, rng_v, sem):
        c = lax.axis_index("c")
        s = lax.axis_index("s")
        pltpu.sync_copy(ranges_hbm, rng_v)
        pltpu.sync_copy(zeros_hbm.at[pl.ds(0, 128)], zero_v)
        nh_mine = (n_half - c + 1) // 2

        def half_body(p, _):
            h = c + 2 * p
            obase = h * r_size
            _zero_acc(acc, zero_v, s, zc)
            plsc.subcore_barrier()
            lo = pl.multiple_of(_scal_from_vmem(rng_v, 2 * h), 256)
            hi = _scal_from_vmem(rng_v, 2 * h + 1)
            nch = (hi - lo) // 256
            nj = (nch - s + 15) // 16

            def chunk_body(i, _):
                base = pl.multiple_of(lo + (s + i * 16) * 256, 256)
                cp_i = pltpu.async_copy(idx2_hbm.at[pl.ds(base // 128, 2)],
                                        idx_v, sem)
                cp_v = pltpu.async_copy(vals_hbm.at[pl.ds(base, 256)],
                                        vals_v, sem)
                cp_i.wait()
                cp_v.wait()
                _adjust_idx(idx_v, obase, r_size)
                pltpu.sync_copy(vals_v.at[pl.ds(0, 128)],
                                acc.at[idx_v.at[0]], add=True)
                pltpu.sync_copy(vals_v.at[pl.ds(128, 128)],
                                acc.at[idx_v.at[1]], add=True)
                return 0

            lax.fori_loop(0, nj, chunk_body, 0)
            plsc.subcore_barrier()
            _copy_out_acc(acc, out_hbm, obase, s, zc)
            plsc.subcore_barrier()
            return 0

        lax.fori_loop(0, nh_mine, half_body, 0)

    return k


def _make_angle_msg(m_rows, d, r_size, n_half):
    zc = r_size // 32

    @functools.partial(
        pl.kernel,
        mesh=_mesh(),
        compiler_params=pltpu.CompilerParams(**_CP),
        out_type=jax.ShapeDtypeStruct((n_half * r_size, d), _f32),
        scratch_types=[
            pltpu.VMEM_SHARED((r_size + 16, d), _f32),
            pltpu.VMEM((256, d), _f32),
            pltpu.VMEM((256, d), _f32),
            pltpu.VMEM((256, d), _f32),
            pltpu.VMEM((2, 256), _i32),
            pltpu.VMEM((2, 256), _i32),
            pltpu.VMEM((2, 2, 128), _i32),
            pltpu.VMEM((128, d), _f32),
            pltpu.VMEM((2 * n_half,), _i32),
            pltpu.SemaphoreType.DMA,
            pltpu.SemaphoreType.DMA,
            pltpu.SemaphoreType.DMA,
            pltpu.SemaphoreType.DMA,
            pltpu.SemaphoreType.DMA,
        ],
        name=f"sc_angle_{m_rows}_{r_size}_{n_half}",
    )
    def k(bond_hbm, t_hbm, g_hbm, exp_hbm, ji_hbm, red2_hbm, ranges_hbm,
          zeros_hbm, out_hbm,
          acc, b_v, t_v, g_v, ei_v, ji_v, red_v, zero_v, rng_v,
          si0, si1, sg0, sg1, ssc):
        c = lax.axis_index("c")
        s = lax.axis_index("s")
        pltpu.sync_copy(ranges_hbm, rng_v)
        pltpu.sync_copy(zeros_hbm.at[pl.ds(0, 128)], zero_v)
        nh_mine = (n_half - c + 1) // 2
        si = (si0, si1)

        def fire_idx(base, b):
            pltpu.async_copy(exp_hbm.at[pl.ds(base, 256)], ei_v.at[b], si[b])
            pltpu.async_copy(ji_hbm.at[pl.ds(base, 256)], ji_v.at[b], si[b])
            pltpu.async_copy(red2_hbm.at[pl.ds(base // 128, 2)],
                             red_v.at[b], si[b])

        def drain_idx(b):
            pltpu.make_async_copy(exp_hbm.at[pl.ds(0, 256)], ei_v.at[b],
                                  si[b]).wait()
            pltpu.make_async_copy(ji_hbm.at[pl.ds(0, 256)], ji_v.at[b],
                                  si[b]).wait()
            pltpu.make_async_copy(red2_hbm.at[pl.ds(0, 2)], red_v.at[b],
                                  si[b]).wait()

        def half_body(p, sc_live):
            h = c + 2 * p
            obase = h * r_size
            _zero_acc(acc, zero_v, s, zc)
            plsc.subcore_barrier()
            lo = pl.multiple_of(_scal_from_vmem(rng_v, 2 * h), 256)
            hi = _scal_from_vmem(rng_v, 2 * h + 1)
            nch = (hi - lo) // 256
            nj = (nch - s + 15) // 16

            @pl.when(nj > 0)
            def _():
                fire_idx(lo + s * 256, 0)

            def chunk_body(i, sc_live):
                b = jax.lax.rem(i, 2)
                for bb in (0, 1):
                    @pl.when(b == bb)
                    def _():
                        process(i, bb)
                return sc_live

            def process(i, bb):
                base = pl.multiple_of(lo + (s + i * 16) * 256, 256)
                drain_idx(bb)

                @pl.when(i + 1 < nj)
                def _():
                    fire_idx(lo + (s + (i + 1) * 16) * 256, 1 - bb)

                @pl.when(i > 0)
                def _():
                    pltpu.make_async_copy(g_hbm.at[pl.ds(0, 256)],
                                          b_v, ssc).wait()
                for off, sgx in ((0, sg0), (128, sg1)):
                    sl = pl.ds(off, 128)
                    pltpu.async_copy(bond_hbm.at[ei_v.at[bb, sl]],
                                     b_v.at[sl], sgx)
                    pltpu.async_copy(t_hbm.at[ji_v.at[bb, sl]], t_v.at[sl],
                                     sgx)
                    pltpu.async_copy(g_hbm.at[pl.ds(base + off, 128)],
                                     g_v.at[sl], sgx)
                _adjust_idx(red_v.at[bb], obase, r_size)
                for off, sgx, rr in ((0, sg0, 0), (128, sg1, 1)):
                    sl = pl.ds(off, 128)
                    for ref in (b_v, t_v, g_v):
                        pltpu.make_async_copy(g_hbm.at[pl.ds(0, 128)],
                                              ref.at[sl], sgx).wait()

                    def mul_body(r, _):
                        for q in range(d // 16):
                            ql = pl.ds(q * 16, 16)
                            b_v[r, ql] = (b_v[r, ql] * g_v[r, ql]
                                          * t_v[r, ql])
                        return 0

                    lax.fori_loop(off, off + 128, mul_body, 0)
                    pltpu.async_copy(b_v.at[sl], acc.at[red_v.at[bb, rr]],
                                     ssc, add=True)

            _ = lax.fori_loop(0, nj, chunk_body, 0)

            @pl.when(nj > 0)
            def _():
                pltpu.make_async_copy(g_hbm.at[pl.ds(0, 256)], b_v,
                                      ssc).wait()
            plsc.subcore_barrier()
            _copy_out_acc(acc, out_hbm, obase, s, zc)
            plsc.subcore_barrier()
            return sc_live

        lax.fori_loop(0, nh_mine, half_body, 0)

    return k


_R_E, _NH_E = 16000, 50
_R_N, _NH_N = 25600, 2


@functools.lru_cache(maxsize=1)
def _sc_kernels():
    return {
        "gather_N": _make_gather(N_PAD, H),
        "g2s_E": _make_gather2sum(E_EDGE, H),
        "scatter_N": _make_scatter_add(E_EDGE, H, _R_N, _NH_N),
        "angle_E": _make_angle_msg(A_ANG, H, _R_E, _NH_E),
    }


def _sorted_ranges(red, r_size, n_half):
    bounds = jnp.arange(n_half + 1, dtype=_i32) * r_size
    ss = jnp.searchsorted(red, bounds).astype(_i32)
    lo = (ss[:-1] // 256) * 256
    hi = jnp.minimum(((ss[1:] + 255) // 256) * 256, red.shape[0])
    return jnp.stack([lo, hi], axis=1).reshape(-1).astype(_i32)


_B_E = 8000
_B_N = 6400


def _row_spec(b, d):
    return pl.BlockSpec((b, d), lambda i: (i, 0))


def _w_spec(*shape):
    return pl.BlockSpec(shape, lambda i: (0,) * len(shape))


def _basis_T(x_ref, dmax):
    x_row = x_ref[...].reshape(1, -1)
    cen = (lax.broadcasted_iota(_i32, (G, 1), 0).astype(_f32)
           * (dmax / (G - 1)))
    return jnp.exp(-((cen - x_row) ** 2) * 5.0)


def _dotT(bT, w):
    return lax.dot_general(bT, w, (((0,), (0,)), ((), ())),
                           preferred_element_type=_f32)


def _pq_body(emb_ref, w0i, w0j, p_ref, q_ref):
    emb = emb_ref[...]
    p_ref[...] = jnp.dot(emb, w0i[...], preferred_element_type=_f32)
    q_ref[...] = jnp.dot(emb, w0j[...], preferred_element_type=_f32)


def _embed_body(dist_ref, gsum_ref, w0r, b0, w1, b1, out_ref):
    rbT = _basis_T(dist_ref, CUTOFF)
    z = _dotT(rbT, w0r[...]) + gsum_ref[...]
    z = _silu(z + b0[...])
    out_ref[...] = _silu(jnp.dot(z, w1[...], preferred_element_type=_f32)
                         + b1[...])


def _basis_body(ang_ref, w_sbf, out_ref):
    out_ref[...] = _dotT(_basis_T(ang_ref, 3.14), w_sbf[...])


def _t_body(bond_ref, wkj, bkj, wim, bim, tkj_ref, tim_ref):
    bond = bond_ref[...]
    tkj_ref[...] = _silu(jnp.dot(bond, wkj[...],
                                 preferred_element_type=_f32) + bkj[...])
    tim_ref[...] = _silu(jnp.dot(bond, wim[...],
                                 preferred_element_type=_f32) + bim[...])


def _bond_update_body(bond_ref, akj_ref, aim_ref, w_out, b_out, wr1, br1,
                      wr2, br2, w_b, b_b, b2_ref, tmp_ref):
    x = bond_ref[...] + akj_ref[...] + aim_ref[...]
    b2 = _silu(jnp.dot(x, w_out[...], preferred_element_type=_f32) + b_out[...])
    b2 = b2 + _silu(jnp.dot(b2, wr1[...], preferred_element_type=_f32)
                    + br1[...])
    b2 = b2 + _silu(jnp.dot(b2, wr2[...], preferred_element_type=_f32)
                    + br2[...])
    b2_ref[...] = b2
    tmp_ref[...] = _silu(jnp.dot(b2, w_b[...], preferred_element_type=_f32)
                         + b_b[...])


def _atom_update_body(atom_ref, agg_ref, hid_ref, w_a, b_a, w_i, w_j,
                      atom_out, hid_out, ai_out, aj_out):
    an = _silu(jnp.dot(atom_ref[...] + agg_ref[...], w_a[...],
                       preferred_element_type=_f32) + b_a[...])
    atom_out[...] = an
    hid_out[...] = hid_ref[...] + an
    ai_out[...] = jnp.dot(an, w_i[...], preferred_element_type=_f32)
    aj_out[...] = jnp.dot(an, w_j[...], preferred_element_type=_f32)


def _bond_atom_body(b2_ref, gsum_ref, w_bb, b_a2b, out_ref):
    out_ref[...] = _silu(jnp.dot(b2_ref[...], w_bb[...],
                                 preferred_element_type=_f32)
                         + gsum_ref[...] + b_a2b[...])


def _readout_body(hg_ref, w0, b0, w1, b1, w2, b2, ow, ob, out_ref):
    h = hg_ref[...]
    h = _silu(jnp.dot(h, w0[...], preferred_element_type=_f32) + b0[...])
    h = _silu(jnp.dot(h, w1[...], preferred_element_type=_f32) + b1[...])
    h = _silu(jnp.dot(h, w2[...], preferred_element_type=_f32) + b2[...])
    out_ref[...] = jnp.dot(h, ow[...], preferred_element_type=_f32) + ob[...]


def _tc_call(body, grid, in_specs, out_specs, out_shapes, name):
    return pl.pallas_call(
        body, grid=(grid,), in_specs=in_specs, out_specs=out_specs,
        out_shape=out_shapes, name=name)


def kernel(atom_features, id_i, id_j, dist, angle_kj, angle_im, kj_exp,
           kj_ji_exp, kj_red, im_exp, im_ji_exp, im_red, reduce_idx, params):
    p = params
    sck = _sc_kernels()
    _gather_N = sck["gather_N"]
    _g2s_E = sck["g2s_E"]
    _scatter_N = sck["scatter_N"]
    _angle_E = sck["angle_E"]
    npad = N_PAD - N_ATOM
    feat_p = jnp.pad(atom_features.astype(_i32), (0, npad))
    reduce_p = jnp.pad(reduce_idx.astype(_i32), (0, npad))
    id_i = id_i.astype(_i32)
    id_j = id_j.astype(_i32)
    id_i2 = id_i.reshape(E_EDGE // 128, 128)
    zeros_z = jnp.zeros((128, H), _f32)
    rng_n = jnp.array([0, E_EDGE, 0, E_EDGE], _i32)

    dist2 = dist.reshape(E_EDGE // _B_E, 1, _B_E)
    akj2 = angle_kj.reshape(A_ANG // _B_E, 1, _B_E)
    aim2 = angle_im.reshape(A_ANG // _B_E, 1, _B_E)

    grid_e = E_EDGE // _B_E
    grid_n = N_PAD // _B_N
    row_e = _row_spec(_B_E, H)
    row_n = _row_spec(_B_N, H)
    scal_e = pl.BlockSpec((1, 1, _B_E), lambda i: (i, 0, 0))
    wh = _w_spec(H, H)
    wb = _w_spec(1, H)
    wg = _w_spec(G, H)
    sde = jax.ShapeDtypeStruct((E_EDGE, H), _f32)
    sdn = jax.ShapeDtypeStruct((N_PAD, H), _f32)

    def b2d(b):
        return b.reshape(1, H)

    atom_emb = _gather_N(p["atom_table"], feat_p)

    w0 = p["emb_W0"]
    pe, qe = _tc_call(_pq_body, grid_n, [row_n, wh, wh],
                      [row_n, row_n], [sdn, sdn], "tc_pq")(
        atom_emb, w0[:H], w0[H:2 * H])
    gsum0 = _g2s_E(pe, qe, id_i, id_j)
    bond = _tc_call(
        _embed_body, grid_e, [scal_e, row_e, wg, wb, wh, wb],
        _row_spec(_B_E, H), sde, "tc_embed")(
        dist2, gsum0, w0[2 * H:], b2d(p["emb_b0"]), p["emb_W1"],
        b2d(p["emb_b1"]))

    atom = atom_emb
    hidden = atom_emb

    kj_red = kj_red.astype(_i32)
    im_red = im_red.astype(_i32)
    rng_kj = _sorted_ranges(kj_red, _R_E, _NH_E)
    rng_im = _sorted_ranges(im_red, _R_E, _NH_E)
    kj_red2 = kj_red.reshape(A_ANG // 128, 128)
    im_red2 = im_red.reshape(A_ANG // 128, 128)

    for lay in p["layers"]:
        g_kj = _tc_call(_basis_body, grid_e, [scal_e, wg],
                        _row_spec(_B_E, H), sde, "tc_basis")(
            akj2, lay["W_sbf_kj"])
        g_im = _tc_call(_basis_body, grid_e, [scal_e, wg],
                        _row_spec(_B_E, H), sde, "tc_basis")(
            aim2, lay["W_sbf_im"])
        t_kj, t_im = _tc_call(
            _t_body, grid_e, [row_e, wh, wb, wh, wb],
            [row_e, row_e], [sde, sde], "tc_tmsg")(
            bond, lay["W_ji_kj"], b2d(lay["b_ji_kj"]),
            lay["W_ji_im"], b2d(lay["b_ji_im"]))

        agg_kj = _angle_E(bond, t_kj, g_kj, kj_exp.astype(_i32),
                          kj_ji_exp.astype(_i32), kj_red2, rng_kj, zeros_z)
        agg_im = _angle_E(bond, t_im, g_im, im_exp.astype(_i32),
                          im_ji_exp.astype(_i32), im_red2, rng_im, zeros_z)

        res = lay["res"]
        b2, tmp = _tc_call(
            _bond_update_body, grid_e,
            [row_e, row_e, row_e, wh, wb, wh, wb, wh, wb, wh, wb],
            [row_e, row_e], [sde, sde], "tc_bond_update")(
            bond, agg_kj, agg_im, lay["W_out"], b2d(lay["b_out"]),
            res[0]["W"], b2d(res[0]["b"]), res[1]["W"], b2d(res[1]["b"]),
            lay["W_b"], b2d(lay["b_b"]))

        agg_a = _scatter_N(tmp, id_i2, rng_n, zeros_z)

        atom, hidden, ai, aj = _tc_call(
            _atom_update_body, grid_n,
            [row_n, row_n, row_n, wh, wb, wh, wh],
            [row_n, row_n, row_n, row_n], [sdn, sdn, sdn, sdn],
            "tc_atom_update")(
            atom, agg_a, hidden, lay["W_a"], b2d(lay["b_a"]),
            lay["W_i"], lay["W_j"])

        gs = _g2s_E(ai, aj, id_i, id_j)
        bond = _tc_call(
            _bond_atom_body, grid_e, [row_e, row_e, wh, wb],
            _row_spec(_B_E, H), sde, "tc_bond_atom")(
            b2, gs, lay["W_bb"], b2d(lay["b_a2b"]))

    hg = _gather_N(hidden, reduce_p)
    ro = p["readout"]
    ow = jnp.pad(p["out_W"], ((0, 0), (0, 128 - p["out_W"].shape[1])))
    ob = jnp.pad(p["out_b"], (0, 128 - p["out_b"].shape[0])).reshape(1, 128)
    out = _tc_call(
        _readout_body, grid_n,
        [row_n, wh, wb, wh, wb, wh, wb, _w_spec(H, 128), _w_spec(1, 128)],
        _row_spec(_B_N, 128), jax.ShapeDtypeStruct((N_PAD, 128), _f32),
        "tc_readout")(
        hg, ro[0]["W"], b2d(ro[0]["b"]), ro[1]["W"], b2d(ro[1]["b"]),
        ro[2]["W"], b2d(ro[2]["b"]), ow, ob)
    return out[:N_ATOM, :p["out_W"].shape[1]]
